# Initial kernel scaffold; baseline (speedup 1.0000x reference)
import jax
import jax.numpy as jnp
from jax.experimental import pallas as pl


def kernel(node_tensors, adj, W_f, bW_f, b_f, W_i, bW_i, b_i, W_u, bW_u, b_u, W_o, bW_o, b_o, L1_W, L1_b, L2_W, L2_b, L3_W, L3_b, L4_W, L4_b):
    raise NotImplementedError("write your pallas kernel here")


# speedup vs baseline: 3.3121x; 3.3121x over previous
import jax
import jax.numpy as jnp
from jax import lax
from jax.experimental import pallas as pl
from jax.experimental.pallas import tpu as pltpu

_N = 2048
_H = 64
_DEPTH = 12


def _encoder_kernel(adj_ref, x_ref, WgH_ref, WgX_ref, bg_ref,
                    L1W_ref, L1b_ref, L2W_ref, L2b_ref, L3W_ref, L3b_ref,
                    L4W_ref, L4b_ref, y_ref, inv_ref, zx_ref, hc_ref):
    adj = adj_ref[0]
    r = jnp.sum(adj, axis=1, keepdims=True)
    inv_ref[...] = jnp.broadcast_to(1.0 / (r + 1e-6), (_N, 2 * _H))
    xs = jnp.dot(adj, x_ref[0], preferred_element_type=jnp.float32)
    xs = xs * inv_ref[:, :_H]
    zx_ref[...] = (jnp.dot(xs, WgX_ref[...], preferred_element_type=jnp.float32)
                   + bg_ref[...])

    def gates(z):
        f = jax.nn.sigmoid(z[:, :_H])
        i = jax.nn.sigmoid(z[:, _H:2 * _H])
        u = jnp.tanh(z[:, 2 * _H:3 * _H])
        o = jax.nn.sigmoid(z[:, 3 * _H:])
        return f, i, u, o

    f, i, u, o = gates(zx_ref[...])
    c = i * u
    h = o * jnp.tanh(c)
    hc_ref[:, :_H] = h
    hc_ref[:, _H:] = c

    def step(_, carry):
        raw = jnp.dot(adj_ref[0], hc_ref[...],
                      preferred_element_type=jnp.float32) * inv_ref[...]
        z = (jnp.dot(raw[:, :_H], WgH_ref[...],
                     preferred_element_type=jnp.float32) + zx_ref[...])
        f, i, u, o = gates(z)
        c_new = i * u + f * raw[:, _H:]
        h_new = o * jnp.tanh(c_new)
        hc_ref[:, :_H] = h_new
        hc_ref[:, _H:] = c_new
        return carry

    lax.fori_loop(0, _DEPTH - 1, step, 0)

    h_root = hc_ref[0:1, :_H]
    y1 = jnp.tanh(jnp.dot(h_root, L1W_ref[...],
                          preferred_element_type=jnp.float32) + L1b_ref[...])
    y2 = (jnp.dot(jax.nn.relu(
              jnp.dot(h_root, L2W_ref[...],
                      preferred_element_type=jnp.float32) + L2b_ref[...]),
          L3W_ref[...], preferred_element_type=jnp.float32) + L3b_ref[...])
    y_ref[0] = jax.nn.relu(
        jnp.dot(y1 + y2, L4W_ref[...],
                preferred_element_type=jnp.float32) + L4b_ref[...])


def kernel(node_tensors, adj, W_f, bW_f, b_f, W_i, bW_i, b_i, W_u, bW_u, b_u,
           W_o, bW_o, b_o, L1_W, L1_b, L2_W, L2_b, L3_W, L3_b, L4_W, L4_b):
    B, N, X = node_tensors.shape
    H = W_f.shape[1]
    Wg = jnp.concatenate([W_f, W_i, W_u, W_o], axis=1)
    WgH = Wg[:H]
    WgX = Wg[H:]
    bg = jnp.concatenate([bW_f + b_f, bW_i + b_i,
                          bW_u + b_u, bW_o + b_o]).reshape(1, 4 * H)

    full = lambda shape: pl.BlockSpec(shape, lambda b: (0,) * len(shape))
    return pl.pallas_call(
        _encoder_kernel,
        grid=(B,),
        in_specs=[
            pl.BlockSpec((1, N, N), lambda b: (b, 0, 0)),
            pl.BlockSpec((1, N, X), lambda b: (b, 0, 0)),
            full((H, 4 * H)),
            full((X, 4 * H)),
            full((1, 4 * H)),
            full((H, H)), full((1, H)),
            full((H, H)), full((1, H)),
            full((H, H)), full((1, H)),
            full((H, H)), full((1, H)),
        ],
        out_specs=pl.BlockSpec((1, 1, H), lambda b: (b, 0, 0)),
        out_shape=jax.ShapeDtypeStruct((B, 1, H), jnp.float32),
        scratch_shapes=[
            pltpu.VMEM((N, 2 * H), jnp.float32),
            pltpu.VMEM((N, 4 * H), jnp.float32),
            pltpu.VMEM((N, 2 * H), jnp.float32),
        ],
    )(adj, node_tensors, WgH, WgX, bg,
      L1_W, L1_b.reshape(1, H), L2_W, L2_b.reshape(1, H),
      L3_W, L3_b.reshape(1, H), L4_W, L4_b.reshape(1, H)).reshape(B, H)
